# baseline (speedup 1.0000x reference)
import functools

import jax
import jax.numpy as jnp
from jax import lax
from jax.experimental import pallas as pl
from jax.experimental.pallas import tpu as pltpu
from jax.experimental.pallas import tpu_sc as plsc

_NUM_CORES = 2
_NUM_SUBCORES = 16
_NUM_WORKERS = _NUM_CORES * _NUM_SUBCORES
_ROWS_PER_CHUNK = 8
_BLOCK_OUT = 512
_GROUP = 16


@functools.cache
def _build(n_rows, n_in, n_out, n_neigh):
  assert n_rows % (_NUM_WORKERS * _ROWS_PER_CHUNK) == 0
  assert n_out >= _BLOCK_OUT and n_out % 8 == 0 and _BLOCK_OUT % 8 == 0
  rounds = n_rows // (_NUM_WORKERS * _ROWS_PER_CHUNK)
  n_blocks = -(-n_out // _BLOCK_OUT)
  n_blocks += n_blocks % 2
  groups = _BLOCK_OUT // _GROUP
  blk_idx = _BLOCK_OUT * n_neigh

  mesh = plsc.VectorSubcoreMesh(
      core_axis_name="c", subcore_axis_name="s",
      num_cores=_NUM_CORES, num_subcores=_NUM_SUBCORES)

  @functools.partial(
      pl.kernel,
      mesh=mesh,
      out_type=jax.ShapeDtypeStruct((n_rows, n_out), jnp.float32),
      scratch_types=[
          pltpu.VMEM((_ROWS_PER_CHUNK * n_in,), jnp.float32),
          pltpu.VMEM((2, n_neigh, _BLOCK_OUT), jnp.int32),
          pltpu.VMEM((2, _ROWS_PER_CHUNK, _BLOCK_OUT), jnp.float32),
          pltpu.SemaphoreType.DMA,
          pltpu.SemaphoreType.DMA,
          pltpu.SemaphoreType.DMA,
          pltpu.SemaphoreType.DMA,
      ],
      compiler_params=pltpu.CompilerParams(
          use_tc_tiling_on_sc=False, needs_layout_passes=False),
  )
  def k(x2_hbm, nbt_hbm, out_hbm, slab, nb, ost, sn0, sn1, so0, so1):
    wid = lax.axis_index("s") * _NUM_CORES + lax.axis_index("c")
    nsem = (sn0, sn1)
    osem = (so0, so1)

    def col_of(ob):
      return jnp.minimum(ob * _BLOCK_OUT, n_out - _BLOCK_OUT)

    def nb_start(ob, par):
      pltpu.async_copy(
          nbt_hbm.at[:, pl.ds(col_of(ob), _BLOCK_OUT)],
          nb.at[par], nsem[par])

    def nb_wait(par):
      pltpu.make_async_copy(
          nbt_hbm.at[:, pl.ds(0, _BLOCK_OUT)], nb.at[par], nsem[par]).wait()

    def out_wait(par):
      pltpu.make_async_copy(
          ost.at[par],
          out_hbm.at[pl.ds(0, _ROWS_PER_CHUNK), pl.ds(0, _BLOCK_OUT)],
          osem[par]).wait()

    def round_body(r, carry):
      rc = wid * rounds + r
      row0 = rc * _ROWS_PER_CHUNK
      nb_start(0, 0)
      pltpu.sync_copy(
          x2_hbm.at[pl.ds(row0 * n_in, _ROWS_PER_CHUNK * n_in)], slab)

      def pair_body(ob2, carry):
        for par in (0, 1):
          ob = ob2 * 2 + par
          col0 = col_of(ob)
          nb_wait(par)

          @pl.when(ob + 1 < n_blocks)
          def _():
            nb_start(ob + 1, 1 - par)

          @pl.when((r > 0) | (ob2 > 0))
          def _():
            out_wait(par)

          def group_body(og, carry):
            g0 = og * _GROUP
            idxs = [nb[par, j, pl.ds(g0, _GROUP)] for j in range(n_neigh)]
            for f in range(_ROWS_PER_CHUNK):
              off = jnp.int32(f * n_in)
              vals = [
                  plsc.load_gather(slab, [idxs[j] + off])
                  for j in range(n_neigh)
              ]
              while len(vals) > 1:
                vals = [jnp.maximum(a, b) for a, b in zip(vals[::2], vals[1::2])]
              ost[par, f, pl.ds(og * _GROUP, _GROUP)] = vals[0]
            return carry

          lax.fori_loop(0, groups, group_body, 0)
          pltpu.async_copy(
              ost.at[par],
              out_hbm.at[pl.ds(row0, _ROWS_PER_CHUNK),
                         pl.ds(col0, _BLOCK_OUT)],
              osem[par])
        return carry

      lax.fori_loop(0, n_blocks // 2, pair_body, 0)
      return carry

    lax.fori_loop(0, rounds, round_body, 0)
    out_wait(0)
    out_wait(1)

  return k


def kernel(x, neighborhood):
  b, f, n_in = x.shape
  n_out, n_neigh = neighborhood.shape
  n_rows = b * f

  x2 = x.reshape(n_rows * n_in)
  nbt = neighborhood.astype(jnp.int32).T
  out2 = _build(n_rows, n_in, n_out, n_neigh)(x2, nbt)
  return out2.reshape(b, f, n_out)


# speedup vs baseline: 1.0930x; 1.0930x over previous
import functools

import jax
import jax.numpy as jnp
from jax import lax
from jax.experimental import pallas as pl
from jax.experimental.pallas import tpu as pltpu
from jax.experimental.pallas import tpu_sc as plsc

_NUM_CORES = 2
_NUM_SUBCORES = 16
_NUM_WORKERS = _NUM_CORES * _NUM_SUBCORES
_ROWS_PER_CHUNK = 8
_BLOCK_OUT = 512
_GROUP = 16


@functools.cache
def _build(n_rows, n_in, n_out, n_neigh):
  assert n_rows % (_NUM_WORKERS * _ROWS_PER_CHUNK) == 0
  assert n_out >= _BLOCK_OUT and n_out % 8 == 0 and _BLOCK_OUT % 8 == 0
  rounds = n_rows // (_NUM_WORKERS * _ROWS_PER_CHUNK)
  n_blocks = -(-n_out // _BLOCK_OUT)
  n_blocks += n_blocks % 2
  groups = _BLOCK_OUT // _GROUP
  blk_idx = _BLOCK_OUT * n_neigh

  mesh = plsc.VectorSubcoreMesh(
      core_axis_name="c", subcore_axis_name="s",
      num_cores=_NUM_CORES, num_subcores=_NUM_SUBCORES)

  @functools.partial(
      pl.kernel,
      mesh=mesh,
      out_type=jax.ShapeDtypeStruct((n_rows, n_out), jnp.float32),
      scratch_types=[
          pltpu.VMEM((_ROWS_PER_CHUNK * n_in,), jnp.float32),
          pltpu.VMEM((2, n_neigh, _BLOCK_OUT), jnp.int32),
          pltpu.VMEM((2, _ROWS_PER_CHUNK, _BLOCK_OUT), jnp.float32),
          pltpu.SemaphoreType.DMA,
          pltpu.SemaphoreType.DMA,
          pltpu.SemaphoreType.DMA,
          pltpu.SemaphoreType.DMA,
      ],
      compiler_params=pltpu.CompilerParams(
          use_tc_tiling_on_sc=False, needs_layout_passes=False),
  )
  def k(x2_hbm, nbt_hbm, out_hbm, slab, nb, ost, sn0, sn1, so0, so1):
    wid = lax.axis_index("s") * _NUM_CORES + lax.axis_index("c")
    nsem = (sn0, sn1)
    osem = (so0, so1)

    def col_of(ob):
      return jnp.minimum(ob * _BLOCK_OUT, n_out - _BLOCK_OUT)

    def nb_start(ob, par):
      pltpu.async_copy(
          nbt_hbm.at[:, pl.ds(col_of(ob), _BLOCK_OUT)],
          nb.at[par], nsem[par])

    def nb_wait(par):
      pltpu.make_async_copy(
          nbt_hbm.at[:, pl.ds(0, _BLOCK_OUT)], nb.at[par], nsem[par]).wait()

    def out_wait(par):
      pltpu.make_async_copy(
          ost.at[par],
          out_hbm.at[pl.ds(0, _ROWS_PER_CHUNK), pl.ds(0, _BLOCK_OUT)],
          osem[par]).wait()

    def round_body(r, carry):
      rc = wid * rounds + r
      row0 = rc * _ROWS_PER_CHUNK
      nb_start(0, 0)
      pltpu.sync_copy(
          x2_hbm.at[pl.ds(row0 * n_in, _ROWS_PER_CHUNK * n_in)], slab)

      def pair_body(ob2, carry):
        for par in (0, 1):
          ob = ob2 * 2 + par
          col0 = col_of(ob)
          nb_wait(par)

          @pl.when(ob + 1 < n_blocks)
          def _():
            nb_start(ob + 1, 1 - par)

          @pl.when((r > 0) | (ob2 > 0))
          def _():
            out_wait(par)

          @plsc.parallel_loop(0, groups)
          def group_body(og):
            g0 = og * _GROUP
            idxs = [nb[par, j, pl.ds(g0, _GROUP)] for j in range(n_neigh)]
            for f in range(_ROWS_PER_CHUNK):
              off = jnp.int32(f * n_in)
              vals = [
                  plsc.load_gather(slab, [idxs[j] + off])
                  for j in range(n_neigh)
              ]
              while len(vals) > 1:
                vals = [jnp.maximum(a, b) for a, b in zip(vals[::2], vals[1::2])]
              ost[par, f, pl.ds(g0, _GROUP)] = vals[0]
          pltpu.async_copy(
              ost.at[par],
              out_hbm.at[pl.ds(row0, _ROWS_PER_CHUNK),
                         pl.ds(col0, _BLOCK_OUT)],
              osem[par])
        return carry

      lax.fori_loop(0, n_blocks // 2, pair_body, 0)
      return carry

    lax.fori_loop(0, rounds, round_body, 0)
    out_wait(0)
    out_wait(1)

  return k


def kernel(x, neighborhood):
  b, f, n_in = x.shape
  n_out, n_neigh = neighborhood.shape
  n_rows = b * f

  x2 = x.reshape(n_rows * n_in)
  nbt = neighborhood.astype(jnp.int32).T
  out2 = _build(n_rows, n_in, n_out, n_neigh)(x2, nbt)
  return out2.reshape(b, f, n_out)


# speedup vs baseline: 1.0946x; 1.0014x over previous
import functools

import jax
import jax.numpy as jnp
from jax import lax
from jax.experimental import pallas as pl
from jax.experimental.pallas import tpu as pltpu
from jax.experimental.pallas import tpu_sc as plsc

_NUM_CORES = 2
_NUM_SUBCORES = 16
_NUM_WORKERS = _NUM_CORES * _NUM_SUBCORES
_ROWS_PER_CHUNK = 8
_BLOCK_OUT = 512
_GROUP = 16


@functools.cache
def _build(n_b, n_f, n_in, n_out, n_neigh):
  n_rows = n_b * n_f
  assert n_f % _ROWS_PER_CHUNK == 0
  assert n_rows % (_NUM_WORKERS * _ROWS_PER_CHUNK) == 0
  assert n_out >= _BLOCK_OUT and n_out % 8 == 0 and _BLOCK_OUT % 8 == 0
  rounds = n_rows // (_NUM_WORKERS * _ROWS_PER_CHUNK)
  n_blocks = -(-n_out // _BLOCK_OUT)
  n_blocks += n_blocks % 2
  groups = _BLOCK_OUT // _GROUP
  blk_idx = _BLOCK_OUT * n_neigh

  mesh = plsc.VectorSubcoreMesh(
      core_axis_name="c", subcore_axis_name="s",
      num_cores=_NUM_CORES, num_subcores=_NUM_SUBCORES)

  @functools.partial(
      pl.kernel,
      mesh=mesh,
      out_type=jax.ShapeDtypeStruct((n_b, n_f, n_out), jnp.float32),
      scratch_types=[
          pltpu.VMEM((_ROWS_PER_CHUNK, n_in), jnp.float32),
          pltpu.VMEM((2, n_neigh, _BLOCK_OUT), jnp.int32),
          pltpu.VMEM((2, _ROWS_PER_CHUNK, _BLOCK_OUT), jnp.float32),
          pltpu.SemaphoreType.DMA,
          pltpu.SemaphoreType.DMA,
          pltpu.SemaphoreType.DMA,
          pltpu.SemaphoreType.DMA,
      ],
      compiler_params=pltpu.CompilerParams(
          use_tc_tiling_on_sc=False, needs_layout_passes=False),
  )
  def k(x2_hbm, nbt_hbm, out_hbm, slab, nb, ost, sn0, sn1, so0, so1):
    wid = lax.axis_index("s") * _NUM_CORES + lax.axis_index("c")
    nsem = (sn0, sn1)
    osem = (so0, so1)

    def col_of(ob):
      return jnp.minimum(ob * _BLOCK_OUT, n_out - _BLOCK_OUT)

    def nb_start(ob, par):
      pltpu.async_copy(
          nbt_hbm.at[:, pl.ds(col_of(ob), _BLOCK_OUT)],
          nb.at[par], nsem[par])

    def nb_wait(par):
      pltpu.make_async_copy(
          nbt_hbm.at[:, pl.ds(0, _BLOCK_OUT)], nb.at[par], nsem[par]).wait()

    def out_wait(par):
      pltpu.make_async_copy(
          ost.at[par],
          out_hbm.at[0, pl.ds(0, _ROWS_PER_CHUNK), pl.ds(0, _BLOCK_OUT)],
          osem[par]).wait()

    chunks_per_b = n_f // _ROWS_PER_CHUNK

    def round_body(r, carry):
      rc = wid * rounds + r
      bb = rc // chunks_per_b
      f0 = (rc % chunks_per_b) * _ROWS_PER_CHUNK
      nb_start(0, 0)
      pltpu.sync_copy(
          x2_hbm.at[bb, pl.ds(f0, _ROWS_PER_CHUNK), :], slab)

      def pair_body(ob2, carry):
        for par in (0, 1):
          ob = ob2 * 2 + par
          col0 = col_of(ob)
          nb_wait(par)

          @pl.when(ob + 1 < n_blocks)
          def _():
            nb_start(ob + 1, 1 - par)

          @pl.when((r > 0) | (ob2 > 0))
          def _():
            out_wait(par)

          @plsc.parallel_loop(0, groups)
          def group_body(og):
            g0 = og * _GROUP
            idxs = [nb[par, j, pl.ds(g0, _GROUP)] for j in range(n_neigh)]
            for f in range(_ROWS_PER_CHUNK):
              rowv = jnp.full((_GROUP,), f, jnp.int32)
              vals = [
                  plsc.load_gather(slab, [rowv, idxs[j]])
                  for j in range(n_neigh)
              ]
              while len(vals) > 1:
                vals = [jnp.maximum(a, b) for a, b in zip(vals[::2], vals[1::2])]
              ost[par, f, pl.ds(g0, _GROUP)] = vals[0]
          pltpu.async_copy(
              ost.at[par],
              out_hbm.at[bb, pl.ds(f0, _ROWS_PER_CHUNK),
                         pl.ds(col0, _BLOCK_OUT)],
              osem[par])
        return carry

      lax.fori_loop(0, n_blocks // 2, pair_body, 0)
      return carry

    lax.fori_loop(0, rounds, round_body, 0)
    out_wait(0)
    out_wait(1)

  return k


def kernel(x, neighborhood):
  b, f, n_in = x.shape
  n_out, n_neigh = neighborhood.shape

  nbt = neighborhood.astype(jnp.int32).T
  return _build(b, f, n_in, n_out, n_neigh)(x, nbt)
